# baseline (speedup 1.0000x reference)
import functools

import jax
import jax.numpy as jnp
from jax import lax
from jax.experimental import pallas as pl
from jax.experimental.pallas import tpu as pltpu
from jax.experimental.pallas import tpu_sc as plsc

_DIM = 128
_LANES = 16
_CH = 400
_NBUF = 2
_K = 4
_SLICES = [(0, 128), (128, 128), (256, 128), (384, 16)]


@functools.cache
def _build(B):
    info = plsc.get_sparse_core_info()
    nc, ns = info.num_cores, info.num_subcores
    nw = nc * ns
    bpw = B // nw
    nchunks = bpw // _CH

    mesh = plsc.VectorSubcoreMesh(core_axis_name="c", subcore_axis_name="s")

    @functools.partial(
        pl.kernel,
        mesh=mesh,
        compiler_params=pltpu.CompilerParams(needs_layout_passes=False),
        out_type=jax.ShapeDtypeStruct((B, _DIM), jnp.float32),
        scratch_types=[
            pltpu.VMEM((bpw,), jnp.int32),
            pltpu.VMEM((_NBUF, _CH, _DIM), jnp.float32),
            pltpu.SemaphoreType.DMA,
            pltpu.SemaphoreType.DMA,
            pltpu.SemaphoreType.DMA,
            pltpu.SemaphoreType.DMA,
        ],
    )
    def gather_k(idx_hbm, w_hbm, out_hbm, idx_all, rows_v, gs0, gs1, ws0,
                 ws1):
        wid = lax.axis_index("s") * nc + lax.axis_index("c")
        base0 = wid * bpw
        gsems = (gs0, gs1)
        wsems = (ws0, ws1)

        pltpu.sync_copy(idx_hbm.at[pl.ds(base0, bpw)], idx_all)

        def gather_start(i, b):
            return [
                pltpu.async_copy(
                    w_hbm.at[idx_all.at[pl.ds(i * _CH + o, n)]],
                    rows_v.at[b, pl.ds(o, n)],
                    gsems[b],
                )
                for o, n in _SLICES
            ]

        def wb_start(i, b):
            pltpu.async_copy(
                rows_v.at[b], out_hbm.at[pl.ds(base0 + i * _CH, _CH)],
                wsems[b])

        def wb_wait(i, b):
            pltpu.make_async_copy(
                rows_v.at[b], out_hbm.at[pl.ds(base0 + i * _CH, _CH)],
                wsems[b]).wait()

        def pad_fix(i, b):
            off = i * _CH
            acc = jnp.zeros((_LANES,), jnp.int32)
            for g in range(_CH // _LANES):
                iv = idx_all[pl.ds(off + g * _LANES, _LANES)]
                acc = acc | (iv == 0).astype(jnp.int32)
            anyz = acc[0]
            for l in range(1, _LANES):
                anyz = anyz | acc[l]

            @pl.when(anyz > 0)
            def _fix():
                def grp(g, c2):
                    iv = idx_all[pl.ds(off + g * _LANES, _LANES)]
                    m = (iv != 0).astype(jnp.float32)
                    rows16 = g * _LANES + lax.iota(jnp.int32, _LANES)
                    for c in range(_DIM):
                        col16 = jnp.full((_LANES,), c, jnp.int32)
                        v = plsc.load_gather(rows_v.at[b], [rows16, col16])
                        plsc.store_scatter(
                            rows_v.at[b], [rows16, col16], v * m)
                    return c2

                lax.fori_loop(0, _CH // _LANES, grp, 0)

        def body(k, carry):
            i0 = k * _K

            @pl.when(i0 >= _NBUF)
            def _d0(i0=i0):
                wb_wait(i0 - _NBUF, 0)

            hs = gather_start(i0, 0)
            for t in range(_K):
                i = i0 + t
                b = t % 2
                nb = 1 - b
                for h in hs:
                    h.wait()
                if t + 1 < _K:
                    @pl.when(i >= 1)
                    def _d(i=i, nb=nb):
                        wb_wait(i - 1, nb)

                    hs = gather_start(i + 1, nb)
                pad_fix(i, b)
                wb_start(i, b)
            return carry

        lax.fori_loop(0, nchunks // _K, body, 0)
        wb_wait(nchunks - 2, 0)
        wb_wait(nchunks - 1, 1)

    return gather_k


def kernel(x, weight):
    B = x.shape[0] * x.shape[1]
    idx = x.reshape(B).astype(jnp.int32)
    out = _build(B)(idx, weight)
    return out.reshape(x.shape + (_DIM,))


# speedup vs baseline: 1.0808x; 1.0808x over previous
import functools

import jax
import jax.numpy as jnp
from jax import lax
from jax.experimental import pallas as pl
from jax.experimental.pallas import tpu as pltpu
from jax.experimental.pallas import tpu_sc as plsc

_DIM = 128
_LANES = 16
_CH = 256
_NBUF = 2
_K = 4
_SLICES = [(0, 128), (128, 128)]


@functools.cache
def _build(B):
    info = plsc.get_sparse_core_info()
    nc, ns = info.num_cores, info.num_subcores
    nw = nc * ns
    bpw = B // nw
    nchunks = bpw // _CH

    mesh = plsc.VectorSubcoreMesh(core_axis_name="c", subcore_axis_name="s")

    @functools.partial(
        pl.kernel,
        mesh=mesh,
        compiler_params=pltpu.CompilerParams(needs_layout_passes=False),
        out_type=jax.ShapeDtypeStruct((B, _DIM), jnp.float32),
        scratch_types=[
            pltpu.VMEM((bpw,), jnp.int32),
            pltpu.VMEM((_NBUF, _CH, _DIM), jnp.float32),
            pltpu.SemaphoreType.DMA,
            pltpu.SemaphoreType.DMA,
            pltpu.SemaphoreType.DMA,
            pltpu.SemaphoreType.DMA,
        ],
    )
    def gather_k(idx_hbm, w_hbm, out_hbm, idx_all, rows_v, gs0, gs1, ws0,
                 ws1):
        wid = lax.axis_index("s") * nc + lax.axis_index("c")
        base0 = wid * bpw
        gsems = (gs0, gs1)
        wsems = (ws0, ws1)

        pltpu.sync_copy(idx_hbm.at[pl.ds(base0, bpw)], idx_all)

        def gather_start(i, b):
            return [
                pltpu.async_copy(
                    w_hbm.at[idx_all.at[pl.ds(i * _CH + o, n)]],
                    rows_v.at[b, pl.ds(o, n)],
                    gsems[b],
                )
                for o, n in _SLICES
            ]

        def wb_start(i, b):
            pltpu.async_copy(
                rows_v.at[b], out_hbm.at[pl.ds(base0 + i * _CH, _CH)],
                wsems[b])

        def wb_wait(i, b):
            pltpu.make_async_copy(
                rows_v.at[b], out_hbm.at[pl.ds(base0 + i * _CH, _CH)],
                wsems[b]).wait()

        def pad_fix(i, b):
            off = i * _CH
            acc = jnp.zeros((_LANES,), jnp.int32)
            for g in range(_CH // _LANES):
                iv = idx_all[pl.ds(off + g * _LANES, _LANES)]
                acc = acc | (iv == 0).astype(jnp.int32)
            anyz = acc[0]
            for l in range(1, _LANES):
                anyz = anyz | acc[l]

            @pl.when(anyz > 0)
            def _fix():
                def grp(g, c2):
                    iv = idx_all[pl.ds(off + g * _LANES, _LANES)]
                    m = (iv != 0).astype(jnp.float32)
                    rows16 = g * _LANES + lax.iota(jnp.int32, _LANES)
                    for c in range(_DIM):
                        col16 = jnp.full((_LANES,), c, jnp.int32)
                        v = plsc.load_gather(rows_v.at[b], [rows16, col16])
                        plsc.store_scatter(
                            rows_v.at[b], [rows16, col16], v * m)
                    return c2

                lax.fori_loop(0, _CH // _LANES, grp, 0)

        def body(k, carry):
            i0 = k * _K

            @pl.when(i0 >= _NBUF)
            def _d0(i0=i0):
                wb_wait(i0 - _NBUF, 0)

            hs = gather_start(i0, 0)
            for t in range(_K):
                i = i0 + t
                b = t % 2
                nb = 1 - b
                for h in hs:
                    h.wait()
                if t + 1 < _K:
                    @pl.when(i >= 1)
                    def _d(i=i, nb=nb):
                        wb_wait(i - 1, nb)

                    hs = gather_start(i + 1, nb)
                pad_fix(i, b)
                wb_start(i, b)
            return carry

        lax.fori_loop(0, nchunks // _K, body, 0)
        wb_wait(nchunks - 2, 0)
        wb_wait(nchunks - 1, 1)

    return gather_k


def kernel(x, weight):
    B = x.shape[0] * x.shape[1]
    idx = x.reshape(B).astype(jnp.int32)
    out = _build(B)(idx, weight)
    return out.reshape(x.shape + (_DIM,))


# speedup vs baseline: 1.0820x; 1.0012x over previous
import functools

import jax
import jax.numpy as jnp
from jax import lax
from jax.experimental import pallas as pl
from jax.experimental.pallas import tpu as pltpu
from jax.experimental.pallas import tpu_sc as plsc

_DIM = 128
_LANES = 16
_CH = 256
_NBUF = 2
_K = 4
_SLICES = [(0, 128), (128, 128)]


@functools.cache
def _build(B):
    info = plsc.get_sparse_core_info()
    nc, ns = info.num_cores, info.num_subcores
    nw = nc * ns
    bpw = B // nw
    nchunks = bpw // _CH

    mesh = plsc.VectorSubcoreMesh(core_axis_name="c", subcore_axis_name="s")

    @functools.partial(
        pl.kernel,
        mesh=mesh,
        compiler_params=pltpu.CompilerParams(
            needs_layout_passes=False, use_tc_tiling_on_sc=False),
        out_type=jax.ShapeDtypeStruct((B, _DIM), jnp.float32),
        scratch_types=[
            pltpu.VMEM((bpw,), jnp.int32),
            pltpu.VMEM((_NBUF, _CH, _DIM), jnp.float32),
            pltpu.SemaphoreType.DMA,
            pltpu.SemaphoreType.DMA,
            pltpu.SemaphoreType.DMA,
            pltpu.SemaphoreType.DMA,
        ],
    )
    def gather_k(idx_hbm, w_hbm, out_hbm, idx_all, rows_v, gs0, gs1, ws0,
                 ws1):
        wid = lax.axis_index("s") * nc + lax.axis_index("c")
        base0 = wid * bpw
        gsems = (gs0, gs1)
        wsems = (ws0, ws1)

        pltpu.sync_copy(idx_hbm.at[pl.ds(base0, bpw)], idx_all)

        def gather_start(i, b):
            return [
                pltpu.async_copy(
                    w_hbm.at[idx_all.at[pl.ds(i * _CH + o, n)]],
                    rows_v.at[b, pl.ds(o, n)],
                    gsems[b],
                )
                for o, n in _SLICES
            ]

        def wb_start(i, b):
            pltpu.async_copy(
                rows_v.at[b], out_hbm.at[pl.ds(base0 + i * _CH, _CH)],
                wsems[b])

        def wb_wait(i, b):
            pltpu.make_async_copy(
                rows_v.at[b], out_hbm.at[pl.ds(base0 + i * _CH, _CH)],
                wsems[b]).wait()

        def pad_fix(i, b):
            off = i * _CH
            acc = jnp.zeros((_LANES,), jnp.int32)
            for g in range(_CH // _LANES):
                iv = idx_all[pl.ds(off + g * _LANES, _LANES)]
                acc = acc | (iv == 0).astype(jnp.int32)
            anyz = acc[0]
            for l in range(1, _LANES):
                anyz = anyz | acc[l]

            @pl.when(anyz > 0)
            def _fix():
                def grp(g, c2):
                    iv = idx_all[pl.ds(off + g * _LANES, _LANES)]
                    m = (iv != 0).astype(jnp.float32)
                    rows16 = g * _LANES + lax.iota(jnp.int32, _LANES)
                    for c in range(_DIM):
                        col16 = jnp.full((_LANES,), c, jnp.int32)
                        v = plsc.load_gather(rows_v.at[b], [rows16, col16])
                        plsc.store_scatter(
                            rows_v.at[b], [rows16, col16], v * m)
                    return c2

                lax.fori_loop(0, _CH // _LANES, grp, 0)

        def body(k, carry):
            i0 = k * _K

            @pl.when(i0 >= _NBUF)
            def _d0(i0=i0):
                wb_wait(i0 - _NBUF, 0)

            hs = gather_start(i0, 0)
            for t in range(_K):
                i = i0 + t
                b = t % 2
                nb = 1 - b
                for h in hs:
                    h.wait()
                if t + 1 < _K:
                    @pl.when(i >= 1)
                    def _d(i=i, nb=nb):
                        wb_wait(i - 1, nb)

                    hs = gather_start(i + 1, nb)
                pad_fix(i, b)
                wb_start(i, b)
            return carry

        lax.fori_loop(0, nchunks // _K, body, 0)
        wb_wait(nchunks - 2, 0)
        wb_wait(nchunks - 1, 1)

    return gather_k


def kernel(x, weight):
    B = x.shape[0] * x.shape[1]
    idx = x.reshape(B).astype(jnp.int32)
    out = _build(B)(idx, weight)
    return out.reshape(x.shape + (_DIM,))
